# baseline (speedup 1.0000x reference)
import functools

import jax
import jax.numpy as jnp
from jax.experimental import pallas as pl
from jax.experimental.pallas import tpu as pltpu

_N = 10000
_E = 320000
_D = 128
_H = 8
_DH = 16
_R = 16
_SCALE = 4.0


def _prep_body(rel_ref, wq_ref, bq_ref, wk_ref, bk_ref, wv_ref, bv_ref,
               qe_ref, wall_ref, ball_ref):
    rel = rel_ref[...]
    qe = qe_ref[...]
    wq = wq_ref[...]
    wk = wk_ref[...]
    wv = wv_ref[...]
    wq_top, wq_bot = wq[:_D], wq[_D:]
    wk_top, wk_bot = wk[:_D], wk[_D:]
    qb = qe @ wq_bot + bq_ref[...]
    kb = qe @ wk_bot + bk_ref[...]
    rel_rep = jnp.broadcast_to(rel[:, None, :], (_R, 16, _D)).reshape(_R * 16, _D)
    a = rel_rep.T
    c_idx = jax.lax.broadcasted_iota(jnp.int32, (_D, _R * 16), 0)
    m_idx = jax.lax.broadcasted_iota(jnp.int32, (_D, _R * 16), 1)
    w2 = jnp.where((c_idx // _DH) == (m_idx % 16), a, 0.0)
    wqr = jnp.dot(wq_top, w2, preferred_element_type=jnp.float32)
    bqr = jnp.dot(qb, w2, preferred_element_type=jnp.float32)
    wall = jnp.concatenate([wq_top, wk_top, wv, wqr], axis=1)
    ball = jnp.concatenate([qb, kb, bv_ref[...], bqr], axis=1)
    wall_ref[...] = wall
    ball_ref[...] = jnp.broadcast_to(ball, (8, 640))


def _proj_body(x_ref, wall_ref, ball_ref, out_ref):
    x = x_ref[...]
    out_ref[...] = (
        jnp.dot(x, wall_ref[...], preferred_element_type=jnp.float32)
        + ball_ref[0][None, :]
    )


def _final_body(msg_ref, seg_ref, wo_ref, bo_ref, out_ref):
    inv = 1.0 / (seg_ref[...] + 1e-8)
    invr = jnp.broadcast_to(inv[:, :, None], (inv.shape[0], _H, _DH))
    invr = invr.reshape(inv.shape[0], _D)
    y = msg_ref[...] * invr
    out_ref[...] = (
        jnp.dot(y, wo_ref[...], preferred_element_type=jnp.float32)
        + bo_ref[0][None, :]
    )


def kernel(node_features, query_embedding, edge_index, edge_type,
           relation_embeddings, Wq, bq, Wk, bk, Wv, bv, Wo, bo):
    wall, ball = pl.pallas_call(
        _prep_body,
        out_shape=[
            jax.ShapeDtypeStruct((_D, 640), jnp.float32),
            jax.ShapeDtypeStruct((8, 640), jnp.float32),
        ],
    )(relation_embeddings, Wq, bq.reshape(1, _D), Wk, bk.reshape(1, _D),
      Wv, bv.reshape(1, _D), query_embedding)

    blk = 2000
    qkvqr = pl.pallas_call(
        _proj_body,
        grid=(_N // blk,),
        in_specs=[
            pl.BlockSpec((blk, _D), lambda i: (i, 0)),
            pl.BlockSpec((_D, 640), lambda i: (0, 0)),
            pl.BlockSpec((8, 640), lambda i: (0, 0)),
        ],
        out_specs=pl.BlockSpec((blk, 640), lambda i: (i, 0)),
        out_shape=jax.ShapeDtypeStruct((_N, 640), jnp.float32),
    )(node_features, wall, ball)

    q = qkvqr[:, :_D]
    k = qkvqr[:, _D:2 * _D]
    v = qkvqr[:, 2 * _D:3 * _D]
    qrw = qkvqr[:, 3 * _D:]

    src = edge_index[0]
    dst = edge_index[1]
    qk = (q[dst] * k[src]).reshape(_E, _H, _DH).sum(-1)
    qr = qrw.reshape(_N * _R, 16)[dst * _R + edge_type, :_H]
    ex = jnp.exp((qk + qr) / _SCALE)
    seg = jax.ops.segment_sum(ex, dst, num_segments=_N)
    msg = jax.ops.segment_sum(
        ex[:, :, None] * v[src].reshape(_E, _H, _DH), dst,
        num_segments=_N).reshape(_N, _D)

    out = pl.pallas_call(
        _final_body,
        grid=(_N // blk,),
        in_specs=[
            pl.BlockSpec((blk, _D), lambda i: (i, 0)),
            pl.BlockSpec((blk, _H), lambda i: (i, 0)),
            pl.BlockSpec((_D, _D), lambda i: (0, 0)),
            pl.BlockSpec((8, _D), lambda i: (0, 0)),
        ],
        out_specs=pl.BlockSpec((blk, _D), lambda i: (i, 0)),
        out_shape=jax.ShapeDtypeStruct((_N, _D), jnp.float32),
    )(msg, seg, Wo, jnp.broadcast_to(bo.reshape(1, _D), (8, _D)))
    return out
